# baseline (speedup 1.0000x reference)
import functools

import jax
import jax.numpy as jnp
from jax import lax
from jax.experimental import pallas as pl
from jax.experimental.pallas import tpu as pltpu
from jax.experimental.pallas import tpu_sc as plsc

ROWS = 16384
COLS = 512
LANES = 16
NUM_GROUPS = COLS // LANES

NUM_CORES = 2
NUM_SUBCORES = 16
NUM_WORKERS = NUM_CORES * NUM_SUBCORES
ROWS_PER_WORKER = ROWS // NUM_WORKERS

CHUNK_ROWS = 64
NUM_CHUNKS = ROWS_PER_WORKER // CHUNK_ROWS


CHUNK_ELEMS = CHUNK_ROWS * COLS


def _permute_body(x_hbm, perm_hbm, out_hbm, perm_v, in_v, out_v):
    wid = lax.axis_index("s") * NUM_CORES + lax.axis_index("c")
    base = wid * ROWS_PER_WORKER * COLS

    pltpu.sync_copy(perm_hbm, perm_v)

    def chunk_body(c, carry):
        elem0 = base + c * CHUNK_ELEMS
        pltpu.sync_copy(x_hbm.at[pl.ds(elem0, CHUNK_ELEMS)], in_v)

        def row_body(r, carry2):
            roff = r * COLS
            for g in range(NUM_GROUPS):
                idx = perm_v[pl.ds(g * LANES, LANES)] + roff
                vals = plsc.load_gather(in_v, [idx])
                out_v[pl.ds(roff + g * LANES, LANES)] = vals
            return carry2

        lax.fori_loop(0, CHUNK_ROWS, row_body, 0, unroll=False)
        pltpu.sync_copy(out_v, out_hbm.at[pl.ds(elem0, CHUNK_ELEMS)])
        return carry

    lax.fori_loop(0, NUM_CHUNKS, chunk_body, 0, unroll=False)


_permute = pl.kernel(
    _permute_body,
    out_type=jax.ShapeDtypeStruct((ROWS * COLS,), jnp.float32),
    mesh=plsc.VectorSubcoreMesh(
        core_axis_name="c", subcore_axis_name="s",
        num_cores=NUM_CORES, num_subcores=NUM_SUBCORES,
    ),
    scratch_types=[
        pltpu.VMEM((COLS,), jnp.int32),
        pltpu.VMEM((CHUNK_ELEMS,), jnp.float32),
        pltpu.VMEM((CHUNK_ELEMS,), jnp.float32),
    ],
    compiler_params=pltpu.CompilerParams(
        use_tc_tiling_on_sc=False, needs_layout_passes=False,
    ),
)


def kernel(x, permutation):
    out_flat = _permute(jnp.reshape(x, (ROWS * COLS,)), permutation)
    return jnp.reshape(out_flat, (ROWS, COLS))


# speedup vs baseline: 2.1434x; 2.1434x over previous
import functools

import jax
import jax.numpy as jnp
from jax import lax
from jax.experimental import pallas as pl
from jax.experimental.pallas import tpu as pltpu
from jax.experimental.pallas import tpu_sc as plsc

ROWS = 16384
COLS = 512
LANES = 16
NUM_GROUPS = COLS // LANES

NUM_CORES = 2
NUM_SUBCORES = 16
NUM_WORKERS = NUM_CORES * NUM_SUBCORES
ROWS_PER_WORKER = ROWS // NUM_WORKERS

CHUNK_ROWS = 64
NUM_CHUNKS = ROWS_PER_WORKER // CHUNK_ROWS


CHUNK_ELEMS = CHUNK_ROWS * COLS


def _permute_body(x_hbm, perm_hbm, out_hbm, perm_v, in_v, out_v):
    wid = lax.axis_index("s") * NUM_CORES + lax.axis_index("c")
    base = wid * ROWS_PER_WORKER * COLS

    pltpu.sync_copy(perm_hbm, perm_v)
    idx_vecs = [perm_v[pl.ds(g * LANES, LANES)] for g in range(NUM_GROUPS)]

    def chunk_body(c, carry):
        elem0 = base + c * CHUNK_ELEMS
        pltpu.sync_copy(x_hbm.at[pl.ds(elem0, CHUNK_ELEMS)], in_v)

        @plsc.parallel_loop(0, CHUNK_ROWS)
        def row_body(r):
            roff = r * COLS
            src = in_v.at[pl.ds(roff, COLS)]
            for g in range(NUM_GROUPS):
                vals = plsc.load_gather(src, [idx_vecs[g]])
                out_v[pl.ds(roff + g * LANES, LANES)] = vals
        pltpu.sync_copy(out_v, out_hbm.at[pl.ds(elem0, CHUNK_ELEMS)])
        return carry

    lax.fori_loop(0, NUM_CHUNKS, chunk_body, 0, unroll=False)


_permute = pl.kernel(
    _permute_body,
    out_type=jax.ShapeDtypeStruct((ROWS * COLS,), jnp.float32),
    mesh=plsc.VectorSubcoreMesh(
        core_axis_name="c", subcore_axis_name="s",
        num_cores=NUM_CORES, num_subcores=NUM_SUBCORES,
    ),
    scratch_types=[
        pltpu.VMEM((COLS,), jnp.int32),
        pltpu.VMEM((CHUNK_ELEMS,), jnp.float32),
        pltpu.VMEM((CHUNK_ELEMS,), jnp.float32),
    ],
    compiler_params=pltpu.CompilerParams(
        use_tc_tiling_on_sc=False, needs_layout_passes=False,
    ),
)


def kernel(x, permutation):
    out_flat = _permute(jnp.reshape(x, (ROWS * COLS,)), permutation)
    return jnp.reshape(out_flat, (ROWS, COLS))
